# baseline (speedup 1.0000x reference)
import jax
import jax.numpy as jnp
from jax import lax
from jax.experimental import pallas as pl
from jax.experimental.pallas import tpu as pltpu
from jax.experimental.pallas import tpu_sc as plsc

MEM = 20000
IMG = (3, 32, 32)
FEAT = 3 * 32 * 32
NCLS = 100
PK = 128
BATCH = 4096
KCHUNK = 512
NSUB = 32
WROWS = BATCH // NSUB
XSUB = 32

_vector_mesh = plsc.VectorSubcoreMesh(
    core_axis_name="core", subcore_axis_name="subcore")


def _kmap_body(idx_col_ref, idx_row_ref, out_ref):
    own = idx_col_ref[...]
    allv = idx_row_ref[...]
    iota = lax.broadcasted_iota(jnp.int32, (KCHUNK, BATCH), 1)
    sel = jnp.where(own == allv, iota, -1)
    out_ref[...] = jnp.max(sel, axis=1, keepdims=True)


def _make_sc_scatter():
    def body(idx_hbm, kmap_hbm, x_hbm, pkin_hbm, obx_ref, opk_ref,
             iw_vmem, kw_vmem, xw_vmem, pkw_vmem):
        core = lax.axis_index("core")
        sub = lax.axis_index("subcore")
        off = (core * 16 + sub) * WROWS

        pltpu.sync_copy(idx_hbm.at[0, pl.ds(off, WROWS)], iw_vmem)
        pltpu.sync_copy(kmap_hbm.at[0, pl.ds(off, WROWS)], kw_vmem)

        pltpu.sync_copy(pkin_hbm.at[kw_vmem], pkw_vmem)
        pltpu.sync_copy(pkw_vmem, opk_ref.at[iw_vmem])

        for k in range(WROWS // XSUB):
            sl = pl.ds(k * XSUB, XSUB)
            pltpu.sync_copy(x_hbm.at[kw_vmem.at[sl]], xw_vmem)
            pltpu.sync_copy(xw_vmem, obx_ref.at[iw_vmem.at[sl]])

    return pl.kernel(
        body,
        out_type=(),
        mesh=_vector_mesh,
        scratch_types=[
            pltpu.VMEM((WROWS,), jnp.int32),
            pltpu.VMEM((WROWS,), jnp.int32),
            pltpu.VMEM((XSUB, FEAT), jnp.float32),
            pltpu.VMEM((WROWS, PK), jnp.int32),
        ],
    )


def kernel(x, y, logits, t, idx, bx, by, bt, blogits):

    logits_bits = jax.lax.bitcast_convert_type(logits, jnp.int32)
    t_col = jnp.full((BATCH, 1), t, dtype=jnp.int32)
    pad_in = jnp.zeros((BATCH, PK - NCLS - 2), jnp.int32)
    pk_in = jnp.concatenate([logits_bits, y[:, None], t_col, pad_in], axis=1)

    blogits_bits = jax.lax.bitcast_convert_type(blogits, jnp.int32)
    pad_buf = jnp.zeros((MEM, PK - NCLS - 2), jnp.int32)
    pk_buf = jnp.concatenate(
        [blogits_bits, by[:, None], bt[:, None], pad_buf], axis=1)

    kmap = pl.pallas_call(
            _kmap_body,
            grid=(BATCH // KCHUNK,),
            in_specs=[
                pl.BlockSpec((KCHUNK, 1), lambda i: (i, 0)),
                pl.BlockSpec((1, BATCH), lambda i: (0, 0)),
            ],
            out_specs=pl.BlockSpec((KCHUNK, 1), lambda i: (i, 0)),
            out_shape=jax.ShapeDtypeStruct((BATCH, 1), jnp.int32),
        )(idx[:, None], idx[None, :])

    obx_ref = jax.new_ref(bx.reshape(MEM, FEAT))
    opk_ref = jax.new_ref(pk_buf)
    _make_sc_scatter()(idx[None, :], kmap.reshape(1, BATCH),
                       x.reshape(BATCH, FEAT), pk_in, obx_ref, opk_ref)
    bx_new = jax.freeze(obx_ref).reshape((MEM,) + IMG)
    opk = jax.freeze(opk_ref)

    blogits_new = jax.lax.bitcast_convert_type(opk[:, :NCLS], jnp.float32)
    by_new = opk[:, NCLS]
    bt_new = opk[:, NCLS + 1]
    return (bx_new, by_new, bt_new, blogits_new)


# speedup vs baseline: 1.0009x; 1.0009x over previous
import jax
import jax.numpy as jnp
from jax import lax
from jax.experimental import pallas as pl
from jax.experimental.pallas import tpu as pltpu
from jax.experimental.pallas import tpu_sc as plsc

MEM = 20000
IMG = (3, 32, 32)
FEAT = 3 * 32 * 32
NCLS = 100
PK = 128
BATCH = 4096
KCHUNK = 512
NSUB = 32
WROWS = BATCH // NSUB
XSUB = 32


def _kmap_body(idx_col_ref, idx_row_ref, out_ref):
    own = idx_col_ref[...]
    allv = idx_row_ref[...]
    iota = lax.broadcasted_iota(jnp.int32, (KCHUNK, BATCH), 1)
    sel = jnp.where(own == allv, iota, -1)
    out_ref[...] = jnp.max(sel, axis=1, keepdims=True)


def _make_sc_scatter():
    def body(idx_hbm, kmap_hbm, x_hbm, pkin_hbm, obx_ref, opk_ref,
             iw_vmem, kw_vmem, xw_vmem, pkw_vmem):
        core = lax.axis_index("core")
        sub = lax.axis_index("subcore")
        off = (core * 16 + sub) * WROWS

        pltpu.sync_copy(idx_hbm.at[0, pl.ds(off, WROWS)], iw_vmem)
        pltpu.sync_copy(kmap_hbm.at[0, pl.ds(off, WROWS)], kw_vmem)

        pltpu.sync_copy(pkin_hbm.at[kw_vmem], pkw_vmem)
        pltpu.sync_copy(pkw_vmem, opk_ref.at[iw_vmem])

        for k in range(WROWS // XSUB):
            sl = pl.ds(k * XSUB, XSUB)
            pltpu.sync_copy(x_hbm.at[kw_vmem.at[sl]], xw_vmem)
            pltpu.sync_copy(xw_vmem, obx_ref.at[iw_vmem.at[sl]])

    return pl.kernel(
        body,
        out_type=(),
        mesh=plsc.VectorSubcoreMesh(
            core_axis_name="core", subcore_axis_name="subcore"),
        scratch_types=[
            pltpu.VMEM((WROWS,), jnp.int32),
            pltpu.VMEM((WROWS,), jnp.int32),
            pltpu.VMEM((XSUB, FEAT), jnp.float32),
            pltpu.VMEM((WROWS, PK), jnp.int32),
        ],
    )


def kernel(x, y, logits, t, idx, bx, by, bt, blogits):

    logits_bits = jax.lax.bitcast_convert_type(logits, jnp.int32)
    t_col = jnp.full((BATCH, 1), t, dtype=jnp.int32)
    pad_in = jnp.zeros((BATCH, PK - NCLS - 2), jnp.int32)
    pk_in = jnp.concatenate([logits_bits, y[:, None], t_col, pad_in], axis=1)

    blogits_bits = jax.lax.bitcast_convert_type(blogits, jnp.int32)
    pad_buf = jnp.zeros((MEM, PK - NCLS - 2), jnp.int32)
    pk_buf = jnp.concatenate(
        [blogits_bits, by[:, None], bt[:, None], pad_buf], axis=1)

    kmap = pl.pallas_call(
            _kmap_body,
            grid=(BATCH // KCHUNK,),
            in_specs=[
                pl.BlockSpec((KCHUNK, 1), lambda i: (i, 0)),
                pl.BlockSpec((1, BATCH), lambda i: (0, 0)),
            ],
            out_specs=pl.BlockSpec((KCHUNK, 1), lambda i: (i, 0)),
            out_shape=jax.ShapeDtypeStruct((BATCH, 1), jnp.int32),
        )(idx[:, None], idx[None, :])

    obx_ref = jax.new_ref(bx.reshape(MEM, FEAT))
    opk_ref = jax.new_ref(pk_buf)
    _make_sc_scatter()(idx[None, :], kmap.reshape(1, BATCH),
                       x.reshape(BATCH, FEAT), pk_in, obx_ref, opk_ref)
    bx_new = jax.freeze(obx_ref).reshape((MEM,) + IMG)
    opk = jax.freeze(opk_ref)

    blogits_new = jax.lax.bitcast_convert_type(opk[:, :NCLS], jnp.float32)
    by_new = opk[:, NCLS]
    bt_new = opk[:, NCLS + 1]
    return (bx_new, by_new, bt_new, blogits_new)
